# Initial kernel scaffold; baseline (speedup 1.0000x reference)
import functools

import jax
import jax.numpy as jnp
from jax import lax
from jax.experimental import pallas as pl
from jax.experimental.pallas import tpu as pltpu
from jax.experimental.pallas import tpu_sc as plsc

VOCAB = 1000000
EMBED = 64
B = 16384
L = 50

NC = 2
NS = 16
NW = NC * NS

TOTAL = B * L
PER_W = TOTAL // NW
CHUNK = 128
K = 5
GROUP = CHUNK * K
NGROUP = PER_W // GROUP
NCHUNK = PER_W // CHUNK


def _embed_body(weight_hbm, x_hbm, out_hbm, idx_v, rows_v, sem):
    wid = lax.axis_index("s") * NC + lax.axis_index("c")
    pltpu.sync_copy(x_hbm.at[wid], idx_v)

    def group(g, _):
        copies = []
        for j in range(K):
            c = g * K + j
            copies.append(
                pltpu.async_copy(
                    weight_hbm.at[idx_v.at[c]],
                    rows_v.at[pl.ds(j * CHUNK, CHUNK)],
                    sem,
                )
            )
        for cp in copies:
            cp.wait()
        pltpu.sync_copy(rows_v, out_hbm.at[wid, pl.ds(g * GROUP, GROUP)])
        return 0

    lax.fori_loop(0, NGROUP, group, 0)


@jax.jit
def kernel(x, weight):
    x_w = x.reshape(NW, NCHUNK, CHUNK).astype(jnp.int32)
    mesh = plsc.VectorSubcoreMesh(core_axis_name="c", subcore_axis_name="s")
    out = pl.kernel(
        _embed_body,
        out_type=jax.ShapeDtypeStruct((NW, PER_W, EMBED), jnp.float32),
        mesh=mesh,
        scratch_types=[
            pltpu.VMEM((NCHUNK, CHUNK), jnp.int32),
            pltpu.VMEM((GROUP, EMBED), jnp.float32),
            pltpu.SemaphoreType.DMA,
        ],
    )(weight, x_w)
    return out.reshape(B, L, EMBED)


# speedup vs baseline: 1.8419x; 1.8419x over previous
import functools

import jax
import jax.numpy as jnp
from jax import lax
from jax.experimental import pallas as pl
from jax.experimental.pallas import tpu as pltpu
from jax.experimental.pallas import tpu_sc as plsc

VOCAB = 1000000
EMBED = 64
B = 16384
L = 50

NC = 2
NS = 16
NW = NC * NS

TOTAL = B * L
PER_W = TOTAL // NW
CHUNK = 128
K = 5
GROUP = CHUNK * K
NGROUP = PER_W // GROUP
NCHUNK = PER_W // CHUNK


def _embed_body(weight_hbm, x_hbm, out_hbm, idx_v, rows_v, sem):
    wid = lax.axis_index("s") * NC + lax.axis_index("c")
    pltpu.sync_copy(x_hbm.at[wid], idx_v)

    def group(g, _):
        copies = []
        for j in range(K):
            c = g * K + j
            copies.append(
                pltpu.async_copy(
                    weight_hbm.at[idx_v.at[c]],
                    rows_v.at[pl.ds(j * CHUNK, CHUNK)],
                    sem,
                )
            )
        for cp in copies:
            cp.wait()
        pltpu.sync_copy(rows_v, out_hbm.at[wid, pl.ds(g * GROUP, GROUP)])
        return 0

    lax.fori_loop(0, NGROUP, group, 0)


@jax.jit
def kernel(x, weight):
    x_w = x.reshape(NW, NCHUNK, CHUNK).astype(jnp.int32)
    mesh = plsc.VectorSubcoreMesh(core_axis_name="c", subcore_axis_name="s")
    out = pl.kernel(
        _embed_body,
        out_type=jax.ShapeDtypeStruct((NW, PER_W, EMBED), jnp.float32),
        mesh=mesh,
        scratch_types=[
            pltpu.VMEM((NCHUNK, CHUNK), jnp.int32),
            pltpu.VMEM((GROUP, EMBED), jnp.float32),
            pltpu.SemaphoreType.DMA,
        ],
        compiler_params=pltpu.CompilerParams(use_tc_tiling_on_sc=False),
    )(weight, x_w)
    return out.reshape(B, L, EMBED)


# speedup vs baseline: 1.8736x; 1.0172x over previous
import jax
import jax.numpy as jnp
from jax import lax
from jax.experimental import pallas as pl
from jax.experimental.pallas import tpu as pltpu
from jax.experimental.pallas import tpu_sc as plsc

VOCAB = 1000000
EMBED = 64
B = 16384
L = 50

NC = 2
NS = 16
NW = NC * NS

TOTAL = B * L
PER_W = TOTAL // NW
CHUNK = 128
K = 5
GROUP = CHUNK * K
NGROUP = PER_W // GROUP
NCHUNK = PER_W // CHUNK


def _embed_body(weight_hbm, x_hbm, out_hbm, idx_v, rows_v, semg, sems):
    wid = lax.axis_index("s") * NC + lax.axis_index("c")
    pltpu.sync_copy(x_hbm.at[wid], idx_v)

    def fire_gathers(g, buf, sem):
        for j in range(K):
            pltpu.async_copy(
                weight_hbm.at[idx_v.at[g * K + j]],
                rows_v.at[buf, pl.ds(j * CHUNK, CHUNK)],
                sem,
            )

    def drain_gathers(buf, sem):
        pltpu.make_async_copy(
            weight_hbm.at[pl.ds(0, GROUP)], rows_v.at[buf], sem
        ).wait()

    def fire_scatter(g, buf, sem):
        pltpu.async_copy(
            rows_v.at[buf], out_hbm.at[wid, pl.ds(g * GROUP, GROUP)], sem
        )

    def wait_scatter(g, buf, sem):
        pltpu.make_async_copy(
            rows_v.at[buf], out_hbm.at[wid, pl.ds(g * GROUP, GROUP)], sem
        ).wait()

    fire_gathers(0, 0, semg.at[0])
    fire_gathers(1, 1, semg.at[1])
    drain_gathers(0, semg.at[0])
    fire_scatter(0, 0, sems.at[0])

    def pair(t, _):
        g = 2 * t + 1
        wait_scatter(g - 1, 0, sems.at[0])
        fire_gathers(g + 1, 0, semg.at[0])
        drain_gathers(1, semg.at[1])
        fire_scatter(g, 1, sems.at[1])
        wait_scatter(g, 1, sems.at[1])
        fire_gathers(g + 2, 1, semg.at[1])
        drain_gathers(0, semg.at[0])
        fire_scatter(g + 1, 0, sems.at[0])
        return 0

    lax.fori_loop(0, (NGROUP - 2) // 2, pair, 0)

    wait_scatter(NGROUP - 2, 0, sems.at[0])
    drain_gathers(1, semg.at[1])
    fire_scatter(NGROUP - 1, 1, sems.at[1])
    wait_scatter(NGROUP - 1, 1, sems.at[1])


@jax.jit
def kernel(x, weight):
    x_w = x.reshape(NW, NCHUNK, CHUNK).astype(jnp.int32)
    mesh = plsc.VectorSubcoreMesh(core_axis_name="c", subcore_axis_name="s")
    out = pl.kernel(
        _embed_body,
        out_type=jax.ShapeDtypeStruct((NW, PER_W, EMBED), jnp.float32),
        mesh=mesh,
        scratch_types=[
            pltpu.VMEM((NCHUNK, CHUNK), jnp.int32),
            pltpu.VMEM((2, GROUP, EMBED), jnp.float32),
            pltpu.SemaphoreType.DMA((2,)),
            pltpu.SemaphoreType.DMA((2,)),
        ],
        compiler_params=pltpu.CompilerParams(use_tc_tiling_on_sc=False),
    )(weight, x_w)
    return out.reshape(B, L, EMBED)
